# baseline (speedup 1.0000x reference)
import functools

import jax
import jax.numpy as jnp
from jax import lax
from jax.experimental import pallas as pl
from jax.experimental.pallas import tpu as pltpu
from jax.experimental.pallas import tpu_sc as plsc

_NUM_PID = 15080
_SCALAR = 10.0
_ROWS = 64
_CHUNK = 1536
_D = 2048

_SC_CORES = 2
_SC_WORKERS = 8
_RPW = _ROWS // _SC_WORKERS


def _lse_kernel(feat_ref, lut_ref, out_ref, s_ref):
    j = pl.program_id(0)
    nc = pl.num_programs(0)

    @pl.when(j == 0)
    def _init():
        s_ref[...] = jnp.zeros_like(s_ref)

    logits = _SCALAR * lax.dot_general(
        feat_ref[...], lut_ref[...], (((1,), (1,)), ((), ())),
        preferred_element_type=jnp.float32)

    col = j * _CHUNK + lax.broadcasted_iota(jnp.int32, (_ROWS, _CHUNK), 1)
    e = jnp.where(col < _NUM_PID, jnp.exp(logits - _SCALAR), 0.0)
    s_ref[...] += e.reshape(_ROWS, _CHUNK // 128, 128).sum(axis=1)

    @pl.when(j == nc - 1)
    def _fin():
        s_tot = s_ref[...].sum(axis=1, keepdims=True)
        out_ref[...] = jnp.log(s_tot) + _SCALAR


@functools.partial(
    pl.kernel,
    mesh=plsc.VectorSubcoreMesh(core_axis_name="c", subcore_axis_name="s"),
    out_type=jax.ShapeDtypeStruct((_ROWS, 16), jnp.float32),
    scratch_types=[
        pltpu.VMEM((_RPW,), jnp.int32),
        pltpu.VMEM((_RPW, _D), jnp.float32),
        pltpu.VMEM((_RPW, _D), jnp.float32),
        pltpu.VMEM((_RPW, 16), jnp.float32),
        pltpu.SemaphoreType.DMA,
    ],
)
def _sc_picked(ids_hbm, feat_hbm, lut_hbm, out_hbm,
               idx_v, feat_v, rows_v, out_v, sem):
    wid = lax.axis_index("s") * _SC_CORES + lax.axis_index("c")

    @pl.when(wid < _SC_WORKERS)
    def _work():
        base = wid * _RPW
        pltpu.sync_copy(ids_hbm.at[pl.ds(base, _RPW)], idx_v)
        pltpu.sync_copy(feat_hbm.at[pl.ds(base, _RPW)], feat_v)
        gather = pltpu.async_copy(lut_hbm.at[idx_v], rows_v, sem)
        gather.wait()
        for r in range(_RPW):
            def body(k, acc):
                a = feat_v[r, pl.ds(k * 16, 16)]
                b = rows_v[r, pl.ds(k * 16, 16)]
                return acc + a * b
            acc = lax.fori_loop(0, _D // 16, body,
                                jnp.zeros((16,), jnp.float32))
            out_v[r, :] = acc
        pltpu.sync_copy(out_v, out_hbm.at[pl.ds(base, _RPW)])


def kernel(features, gt_labels, lut):
    pids = gt_labels.reshape(-1, gt_labels.shape[-1])[:, -1].astype(jnp.int32)
    mask = pids > -1
    safe = jnp.where(mask, pids, 0)

    nc = pl.cdiv(_NUM_PID, _CHUNK)
    lse = pl.pallas_call(
        _lse_kernel,
        grid=(nc,),
        in_specs=[
            pl.BlockSpec((_ROWS, _D), lambda j: (0, 0)),
            pl.BlockSpec((_CHUNK, _D), lambda j: (j, 0)),
        ],
        out_specs=pl.BlockSpec((_ROWS, 1), lambda j: (0, 0)),
        out_shape=jax.ShapeDtypeStruct((_ROWS, 1), jnp.float32),
        scratch_shapes=[pltpu.VMEM((_ROWS, 128), jnp.float32)],
    )(features, lut)[:, 0]

    picked = _sc_picked(safe, features, lut).sum(axis=1) * _SCALAR

    per_row = jnp.where(mask, lse - picked, 0.0)
    return jnp.sum(per_row) / jnp.sum(mask.astype(jnp.float32))


# speedup vs baseline: 1.4532x; 1.4532x over previous
import jax
import jax.numpy as jnp
from jax.experimental import pallas as pl
from jax.experimental.pallas import tpu as pltpu

_NUM_PID = 15080
_SCALAR = 10.0
_ROWS = 64
_CHUNK = 1536


def _oim_kernel(ids_ref, feat_ref, lut_ref, out_ref, s_ref, p_ref):
    j = pl.program_id(0)
    nc = pl.num_programs(0)

    @pl.when(j == 0)
    def _init():
        s_ref[...] = jnp.zeros_like(s_ref)
        p_ref[...] = jnp.zeros_like(p_ref)

    logits = _SCALAR * jax.lax.dot_general(
        feat_ref[...], lut_ref[...], (((1,), (1,)), ((), ())),
        preferred_element_type=jnp.float32)

    base = j * _CHUNK
    col = base + jax.lax.broadcasted_iota(jnp.int32, (_ROWS, _CHUNK), 1)

    e = jnp.where(col < _NUM_PID, jnp.exp(logits - _SCALAR), 0.0)
    s_ref[...] += e.reshape(_ROWS, _CHUNK // 128, 128).sum(axis=1)

    pids = ids_ref[:, :1]
    row_ok = pids > -1
    safe = jnp.where(row_ok, pids, 0)
    pick = jnp.where(col == safe, logits, 0.0)
    p_ref[...] += pick.reshape(_ROWS, _CHUNK // 128, 128).sum(axis=1)

    @pl.when(j == nc - 1)
    def _fin():
        s_tot = s_ref[...].sum(axis=1, keepdims=True)
        p_tot = p_ref[...].sum(axis=1, keepdims=True)
        lse = jnp.log(s_tot) + _SCALAR
        per_row = jnp.where(row_ok, lse - p_tot, 0.0)
        cnt = jnp.sum(row_ok.astype(jnp.float32))
        out_ref[0, 0] = jnp.sum(per_row) / cnt


def kernel(features, gt_labels, lut):
    pids = gt_labels.reshape(-1, gt_labels.shape[-1])[:, -1].astype(jnp.int32)
    ids2d = jnp.broadcast_to(pids[:, None], (_ROWS, 128))
    nc = pl.cdiv(_NUM_PID, _CHUNK)
    loss = pl.pallas_call(
        _oim_kernel,
        grid=(nc,),
        in_specs=[
            pl.BlockSpec((_ROWS, 128), lambda j: (0, 0)),
            pl.BlockSpec((_ROWS, features.shape[1]), lambda j: (0, 0)),
            pl.BlockSpec((_CHUNK, lut.shape[1]), lambda j: (j, 0)),
        ],
        out_specs=pl.BlockSpec(memory_space=pltpu.SMEM),
        out_shape=jax.ShapeDtypeStruct((1, 1), jnp.float32),
        scratch_shapes=[
            pltpu.VMEM((_ROWS, 128), jnp.float32),
            pltpu.VMEM((_ROWS, 128), jnp.float32),
        ],
    )(ids2d, features, lut)
    return loss[0, 0]
